# Initial kernel scaffold; baseline (speedup 1.0000x reference)
import jax
import jax.numpy as jnp
from jax.experimental import pallas as pl


def kernel(eu, ei, et, ew, u_iw_j, u_iw_w, u_tw_j, u_tw_w, i_uw_j, i_uw_w, i_tw_j, i_tw_w, t_uw_j, t_uw_w, t_iw_j, t_iw_w, W1_user, W2_user, b_user, v_user, W1_item, W2_item, b_item, v_item, W1_tag, W2_tag, b_tag, v_tag, U, q, p):
    raise NotImplementedError("write your pallas kernel here")


# speedup vs baseline: 3.3570x; 3.3570x over previous
import functools

import jax
import jax.numpy as jnp
from jax import lax
from jax.experimental import pallas as pl
from jax.experimental.pallas import tpu as pltpu
from jax.experimental.pallas import tpu_sc as plsc

N = 10000
K = 16
F = 128
A = 128
DW = 16
NK = N * K
NWORK = 32
CHUNK = 128
NCHUNKS = NK // CHUNK
FULL_T = NCHUNKS // NWORK
LEFT = NCHUNKS - FULL_T * NWORK

_HI = jax.lax.Precision.HIGHEST


def _sc_gather_body(tab_hbm, eww_hbm, vj_hbm, vw_hbm, outj_hbm, outw_hbm,
                    idxj_v, idxw_v, rowsj_v, rowsw_v, semj, semw):
    c = lax.axis_index("c")
    s = lax.axis_index("s")
    wid = s * 2 + c

    def do_chunk(cid):
        base = cid * CHUNK
        pltpu.sync_copy(vj_hbm.at[pl.ds(base, CHUNK)], idxj_v)
        pltpu.sync_copy(vw_hbm.at[pl.ds(base, CHUNK)], idxw_v)
        cpj = pltpu.async_copy(tab_hbm.at[idxj_v], rowsj_v, semj)
        cpw = pltpu.async_copy(eww_hbm.at[idxw_v], rowsw_v, semw)
        cpj.wait()
        cpw.wait()
        pltpu.sync_copy(rowsj_v, outj_hbm.at[pl.ds(base, CHUNK)])
        pltpu.sync_copy(rowsw_v, outw_hbm.at[pl.ds(base, CHUNK)])

    def body(t, carry):
        do_chunk(wid + t * NWORK)
        return carry

    lax.fori_loop(0, FULL_T, body, 0)

    @pl.when(wid < LEFT)
    def _():
        do_chunk(FULL_T * NWORK + wid)


@functools.partial(jax.jit, static_argnames=())
def _sc_gather(tab, eww, vj, vw):
    return pl.kernel(
        _sc_gather_body,
        mesh=plsc.VectorSubcoreMesh(core_axis_name="c", subcore_axis_name="s"),
        compiler_params=pltpu.CompilerParams(use_tc_tiling_on_sc=False),
        out_type=(
            jax.ShapeDtypeStruct((NK, F), jnp.float32),
            jax.ShapeDtypeStruct((NK, DW), jnp.float32),
        ),
        scratch_types=[
            pltpu.VMEM((CHUNK,), jnp.int32),
            pltpu.VMEM((CHUNK,), jnp.int32),
            pltpu.VMEM((CHUNK, F), jnp.float32),
            pltpu.VMEM((CHUNK, DW), jnp.float32),
            pltpu.SemaphoreType.DMA,
            pltpu.SemaphoreType.DMA,
        ],
    )(tab, eww, vj, vw)


_B1 = 400


def _atten1_tc_body(ev_ref, ejn_ref, ewn_ref, w1e_ref, w1w_ref, w2_ref,
                    b_ref, v_ref, out_ref):
    ev = ev_ref[...]
    ejn = ejn_ref[...]
    ewn = ewn_ref[...]
    h = jnp.dot(ev, w1e_ref[...], precision=_HI) + b_ref[...]
    hj = jnp.dot(ejn, w2_ref[...], precision=_HI)
    hw = jnp.dot(ewn, w1w_ref[...], precision=_HI)
    av = (hj + hw).reshape(_B1, K, A) + h[:, None, :]
    x = jnp.sum(jnp.maximum(av, 0.0) * v_ref[...].reshape(1, 1, A), axis=-1)
    m = jnp.max(x, axis=1, keepdims=True)
    e = jnp.exp(x - m)
    a = e / jnp.sum(e, axis=1, keepdims=True)
    out_ref[...] = jnp.sum(a[:, :, None] * ejn.reshape(_B1, K, F), axis=1)


def _atten1_tc(ev, ejn, ewn, w1e, w1w, w2, b, v):
    grid = (N // _B1,)
    return pl.pallas_call(
        _atten1_tc_body,
        grid=grid,
        in_specs=[
            pl.BlockSpec((_B1, F), lambda i: (i, 0)),
            pl.BlockSpec((_B1 * K, F), lambda i: (i, 0)),
            pl.BlockSpec((_B1 * K, DW), lambda i: (i, 0)),
            pl.BlockSpec((F, A), lambda i: (0, 0)),
            pl.BlockSpec((DW, A), lambda i: (0, 0)),
            pl.BlockSpec((F, A), lambda i: (0, 0)),
            pl.BlockSpec((1, A), lambda i: (0, 0)),
            pl.BlockSpec((1, A), lambda i: (0, 0)),
        ],
        out_specs=pl.BlockSpec((_B1, F), lambda i: (i, 0)),
        out_shape=jax.ShapeDtypeStruct((N, F), jnp.float32),
    )(ev, ejn, ewn, w1e, w1w, w2, b, v)


def _atten2_tc_body(u_ref, i_ref, t_ref, U_ref, q_ref, p_ref, out_ref):
    u = u_ref[...]
    i = i_ref[...]
    t = t_ref[...]
    Um = U_ref[...]
    q = q_ref[...]
    p = p_ref[...]
    xu = jnp.dot(u, Um, precision=_HI) + q
    xi = jnp.dot(i, Um, precision=_HI) + q
    xt = jnp.dot(t, Um, precision=_HI) + q
    su = jnp.sum(jnp.maximum(xu, 0.0) * p, axis=-1, keepdims=True)
    si = jnp.sum(jnp.maximum(xi, 0.0) * p, axis=-1, keepdims=True)
    st = jnp.sum(jnp.maximum(xt, 0.0) * p, axis=-1, keepdims=True)
    x = jnp.concatenate([su, si, st], axis=1)
    m = jnp.max(x, axis=1, keepdims=True)
    e = jnp.exp(x - m)
    a = e / jnp.sum(e, axis=1, keepdims=True)
    out_ref[...] = (a[:, 0:1] * u + a[:, 1:2] * i + a[:, 2:3] * t)


def _atten2_tc(u, i, t, U, q, p):
    grid = (N // _B1,)
    blk = pl.BlockSpec((_B1, F), lambda g: (g, 0))
    return pl.pallas_call(
        _atten2_tc_body,
        grid=grid,
        in_specs=[blk, blk, blk,
                  pl.BlockSpec((F, A), lambda g: (0, 0)),
                  pl.BlockSpec((1, A), lambda g: (0, 0)),
                  pl.BlockSpec((1, A), lambda g: (0, 0))],
        out_specs=blk,
        out_shape=jax.ShapeDtypeStruct((N, F), jnp.float32),
    )(u, i, t, U, q, p)


def kernel(eu, ei, et, ew, u_iw_j, u_iw_w, u_tw_j, u_tw_w, i_uw_j, i_uw_w,
           i_tw_j, i_tw_w, t_uw_j, t_uw_w, t_iw_j, t_iw_w, W1_user, W2_user,
           b_user, v_user, W1_item, W2_item, b_item, v_item, W1_tag, W2_tag,
           b_tag, v_tag, U, q, p):
    zrow = jnp.zeros((1, F), jnp.float32)
    eu_p = jnp.concatenate([zrow, eu], axis=0)
    ei_p = jnp.concatenate([zrow, ei], axis=0)
    et_p = jnp.concatenate([zrow, et], axis=0)
    ew_p = jnp.concatenate([jnp.zeros((1, DW), jnp.float32), ew], axis=0)

    def atten1(ev, ejtab, vj, vw, W1, W2, b, v):
        ejn, ewn = _sc_gather(ejtab, ew_p, vj.reshape(-1), vw.reshape(-1))
        return _atten1_tc(ev, ejn, ewn, W1[:F], W1[F:], W2, b, v)

    eu_iN = atten1(eu, ei_p, u_iw_j, u_iw_w, W1_item, W2_item, b_item, v_item)
    eu_tN = atten1(eu, et_p, u_tw_j, u_tw_w, W1_tag, W2_tag, b_tag, v_tag)
    ei_uN = atten1(ei, eu_p, i_uw_j, i_uw_w, W1_user, W2_user, b_user, v_user)
    ei_tN = atten1(ei, et_p, i_tw_j, i_tw_w, W1_tag, W2_tag, b_tag, v_tag)
    et_uN = atten1(et, eu_p, t_uw_j, t_uw_w, W1_user, W2_user, b_user, v_user)
    et_iN = atten1(et, ei_p, t_iw_j, t_iw_w, W1_item, W2_item, b_item, v_item)

    euN = _atten2_tc(eu, eu_iN, eu_tN, U, q, p)
    eiN = _atten2_tc(ei_uN, ei, ei_tN, U, q, p)
    etN = _atten2_tc(et_uN, et_iN, et, U, q, p)
    return (euN, eiN, etN)


# speedup vs baseline: 3.5273x; 1.0507x over previous
import jax
import jax.numpy as jnp
from jax import lax
from jax.experimental import pallas as pl
from jax.experimental.pallas import tpu as pltpu
from jax.experimental.pallas import tpu_sc as plsc

N = 10000
K = 16
F = 128
A = 128
DW = 16
NK = N * K
NCALLS = 6
NKALL = NCALLS * NK
NWORK = 32
RPW = NKALL // NWORK
CHUNK = 128
NBUF = 3
NCH = RPW // CHUNK
TAIL = RPW - NCH * CHUNK
GROUPS = NCH // NBUF

_HI = jax.lax.Precision.HIGHEST


def _sc_gather_body(tabj_hbm, tabw_hbm, vj_hbm, vw_hbm, outj_hbm, outw_hbm,
                    idxj_v, idxw_v, rj0, rj1, rj2, rw0, rw1, rw2,
                    sg0, sg1, sg2, sw0, sw1, sw2):
    c = lax.axis_index("c")
    s = lax.axis_index("s")
    wid = s * 2 + c
    rbase = wid * RPW
    rj = [rj0, rj1, rj2]
    rw = [rw0, rw1, rw2]
    sg = [sg0, sg1, sg2]
    sw = [sw0, sw1, sw2]

    pltpu.sync_copy(vj_hbm.at[pl.ds(rbase, RPW)], idxj_v)
    pltpu.sync_copy(vw_hbm.at[pl.ds(rbase, RPW)], idxw_v)

    def fire_gather(b, ch):
        off = ch * CHUNK
        pltpu.async_copy(tabj_hbm.at[idxj_v.at[pl.ds(off, CHUNK)]], rj[b], sg[b])
        pltpu.async_copy(tabw_hbm.at[idxw_v.at[pl.ds(off, CHUNK)]], rw[b], sg[b])

    def wait_gather(b):
        pltpu.make_async_copy(tabj_hbm.at[idxj_v.at[pl.ds(0, CHUNK)]], rj[b], sg[b]).wait()
        pltpu.make_async_copy(tabw_hbm.at[idxw_v.at[pl.ds(0, CHUNK)]], rw[b], sg[b]).wait()

    def fire_wb(b, ch):
        gb = rbase + ch * CHUNK
        pltpu.async_copy(rj[b], outj_hbm.at[pl.ds(gb, CHUNK)], sw[b])
        pltpu.async_copy(rw[b], outw_hbm.at[pl.ds(gb, CHUNK)], sw[b])

    def wait_wb(b):
        pltpu.make_async_copy(rj[b], outj_hbm.at[pl.ds(0, CHUNK)], sw[b]).wait()
        pltpu.make_async_copy(rw[b], outw_hbm.at[pl.ds(0, CHUNK)], sw[b]).wait()

    for b in range(NBUF):
        fire_gather(b, b)

    def group(g, carry):
        ch0 = g * NBUF
        for b in range(NBUF):
            wait_gather(b)
            fire_wb(b, ch0 + b)
        for b in range(NBUF):
            @pl.when(g < GROUPS - 1)
            def _(b=b):
                wait_wb(b)
                fire_gather(b, ch0 + NBUF + b)
        return carry

    lax.fori_loop(0, GROUPS, group, 0)
    for b in range(NBUF):
        wait_wb(b)

    toff = NCH * CHUNK
    tj = rj[0].at[pl.ds(0, TAIL)]
    tw = rw[0].at[pl.ds(0, TAIL)]
    pltpu.async_copy(tabj_hbm.at[idxj_v.at[pl.ds(toff, TAIL)]], tj, sg0)
    pltpu.async_copy(tabw_hbm.at[idxw_v.at[pl.ds(toff, TAIL)]], tw, sg0)
    pltpu.make_async_copy(tabj_hbm.at[idxj_v.at[pl.ds(toff, TAIL)]], tj, sg0).wait()
    pltpu.make_async_copy(tabw_hbm.at[idxw_v.at[pl.ds(toff, TAIL)]], tw, sg0).wait()
    pltpu.sync_copy(tj, outj_hbm.at[pl.ds(rbase + toff, TAIL)])
    pltpu.sync_copy(tw, outw_hbm.at[pl.ds(rbase + toff, TAIL)])


def _sc_gather(tabj, tabw, vj, vw):
    return pl.kernel(
        _sc_gather_body,
        mesh=plsc.VectorSubcoreMesh(core_axis_name="c", subcore_axis_name="s"),
        compiler_params=pltpu.CompilerParams(use_tc_tiling_on_sc=False),
        out_type=(
            jax.ShapeDtypeStruct((NKALL, F), jnp.float32),
            jax.ShapeDtypeStruct((NKALL, DW), jnp.float32),
        ),
        scratch_types=[
            pltpu.VMEM((RPW,), jnp.int32),
            pltpu.VMEM((RPW,), jnp.int32),
            pltpu.VMEM((CHUNK, F), jnp.float32),
            pltpu.VMEM((CHUNK, F), jnp.float32),
            pltpu.VMEM((CHUNK, F), jnp.float32),
            pltpu.VMEM((CHUNK, DW), jnp.float32),
            pltpu.VMEM((CHUNK, DW), jnp.float32),
            pltpu.VMEM((CHUNK, DW), jnp.float32),
            pltpu.SemaphoreType.DMA,
            pltpu.SemaphoreType.DMA,
            pltpu.SemaphoreType.DMA,
            pltpu.SemaphoreType.DMA,
            pltpu.SemaphoreType.DMA,
            pltpu.SemaphoreType.DMA,
        ],
    )(tabj, tabw, vj, vw)


_B1 = 400


def _atten1_tc_body(ev_ref, ejn_ref, ewn_ref, w1e_ref, w1w_ref, w2_ref,
                    b_ref, v_ref, out_ref):
    ev = ev_ref[...]
    ejn = ejn_ref[...]
    ewn = ewn_ref[...]
    h = jnp.dot(ev, w1e_ref[...], precision=_HI) + b_ref[...]
    hj = jnp.dot(ejn, w2_ref[...], precision=_HI)
    hw = jnp.dot(ewn, w1w_ref[...], precision=_HI)
    av = (hj + hw).reshape(_B1, K, A) + h[:, None, :]
    x = jnp.sum(jnp.maximum(av, 0.0) * v_ref[...].reshape(1, 1, A), axis=-1)
    m = jnp.max(x, axis=1, keepdims=True)
    e = jnp.exp(x - m)
    a = e / jnp.sum(e, axis=1, keepdims=True)
    out_ref[...] = jnp.sum(a[:, :, None] * ejn.reshape(_B1, K, F), axis=1)


def _atten1_tc(call_idx, ev, ejn_all, ewn_all, w1e, w1w, w2, b, v):
    grid = (N // _B1,)
    boff = call_idx * (NK // (_B1 * K))
    return pl.pallas_call(
        _atten1_tc_body,
        grid=grid,
        in_specs=[
            pl.BlockSpec((_B1, F), lambda i: (i, 0)),
            pl.BlockSpec((_B1 * K, F), lambda i, o=boff: (o + i, 0)),
            pl.BlockSpec((_B1 * K, DW), lambda i, o=boff: (o + i, 0)),
            pl.BlockSpec((F, A), lambda i: (0, 0)),
            pl.BlockSpec((DW, A), lambda i: (0, 0)),
            pl.BlockSpec((F, A), lambda i: (0, 0)),
            pl.BlockSpec((1, A), lambda i: (0, 0)),
            pl.BlockSpec((1, A), lambda i: (0, 0)),
        ],
        out_specs=pl.BlockSpec((_B1, F), lambda i: (i, 0)),
        out_shape=jax.ShapeDtypeStruct((N, F), jnp.float32),
    )(ev, ejn_all, ewn_all, w1e, w1w, w2, b, v)


def _atten2_tc_body(u_ref, i_ref, t_ref, U_ref, q_ref, p_ref, out_ref):
    u = u_ref[...]
    i = i_ref[...]
    t = t_ref[...]
    Um = U_ref[...]
    q = q_ref[...]
    p = p_ref[...]
    xu = jnp.dot(u, Um, precision=_HI) + q
    xi = jnp.dot(i, Um, precision=_HI) + q
    xt = jnp.dot(t, Um, precision=_HI) + q
    su = jnp.sum(jnp.maximum(xu, 0.0) * p, axis=-1, keepdims=True)
    si = jnp.sum(jnp.maximum(xi, 0.0) * p, axis=-1, keepdims=True)
    st = jnp.sum(jnp.maximum(xt, 0.0) * p, axis=-1, keepdims=True)
    x = jnp.concatenate([su, si, st], axis=1)
    m = jnp.max(x, axis=1, keepdims=True)
    e = jnp.exp(x - m)
    a = e / jnp.sum(e, axis=1, keepdims=True)
    out_ref[...] = (a[:, 0:1] * u + a[:, 1:2] * i + a[:, 2:3] * t)


def _atten2_tc(u, i, t, U, q, p):
    grid = (N // _B1,)
    blk = pl.BlockSpec((_B1, F), lambda g: (g, 0))
    return pl.pallas_call(
        _atten2_tc_body,
        grid=grid,
        in_specs=[blk, blk, blk,
                  pl.BlockSpec((F, A), lambda g: (0, 0)),
                  pl.BlockSpec((1, A), lambda g: (0, 0)),
                  pl.BlockSpec((1, A), lambda g: (0, 0))],
        out_specs=blk,
        out_shape=jax.ShapeDtypeStruct((N, F), jnp.float32),
    )(u, i, t, U, q, p)


def kernel(eu, ei, et, ew, u_iw_j, u_iw_w, u_tw_j, u_tw_w, i_uw_j, i_uw_w,
           i_tw_j, i_tw_w, t_uw_j, t_uw_w, t_iw_j, t_iw_w, W1_user, W2_user,
           b_user, v_user, W1_item, W2_item, b_item, v_item, W1_tag, W2_tag,
           b_tag, v_tag, U, q, p):
    zrow = jnp.zeros((1, F), jnp.float32)
    tabj = jnp.concatenate([zrow, eu, zrow, ei, zrow, et], axis=0)
    tabw = jnp.concatenate([jnp.zeros((1, DW), jnp.float32), ew], axis=0)

    tmap = (1, 2, 0, 2, 0, 1)
    vjs = (u_iw_j, u_tw_j, i_uw_j, i_tw_j, t_uw_j, t_iw_j)
    vws = (u_iw_w, u_tw_w, i_uw_w, i_tw_w, t_uw_w, t_iw_w)
    vj_all = jnp.concatenate(
        [v.reshape(-1) + jnp.int32(tm * (N + 1)) for v, tm in zip(vjs, tmap)])
    vw_all = jnp.concatenate([v.reshape(-1) for v in vws])

    ejn_all, ewn_all = _sc_gather(tabj, tabw, vj_all, vw_all)

    def atten1(c, ev, W1, W2, b, v):
        return _atten1_tc(c, ev, ejn_all, ewn_all, W1[:F], W1[F:], W2, b, v)

    eu_iN = atten1(0, eu, W1_item, W2_item, b_item, v_item)
    eu_tN = atten1(1, eu, W1_tag, W2_tag, b_tag, v_tag)
    ei_uN = atten1(2, ei, W1_user, W2_user, b_user, v_user)
    ei_tN = atten1(3, ei, W1_tag, W2_tag, b_tag, v_tag)
    et_uN = atten1(4, et, W1_user, W2_user, b_user, v_user)
    et_iN = atten1(5, et, W1_item, W2_item, b_item, v_item)

    euN = _atten2_tc(eu, eu_iN, eu_tN, U, q, p)
    eiN = _atten2_tc(ei_uN, ei, ei_tN, U, q, p)
    etN = _atten2_tc(et_uN, et_iN, et, U, q, p)
    return (euN, eiN, etN)


# speedup vs baseline: 5.5626x; 1.5770x over previous
import jax
import jax.numpy as jnp
from jax import lax
from jax.experimental import pallas as pl
from jax.experimental.pallas import tpu as pltpu
from jax.experimental.pallas import tpu_sc as plsc

N = 10000
K = 16
F = 128
A = 128
DW = 16
NK = N * K
NCALLS = 6
NKALL = NCALLS * NK
NWORK = 32
RPW = NKALL // NWORK
CHUNK = 128
NBUF = 3
NCH = RPW // CHUNK
TAIL = RPW - NCH * CHUNK
GROUPS = NCH // NBUF

_HI = jax.lax.Precision.HIGHEST


def _sc_gather_body(tabj_hbm, tabw_hbm, vj_hbm, vw_hbm, outj_hbm, outw_hbm,
                    idxj_v, idxw_v, rj0, rj1, rj2, rw0, rw1, rw2,
                    sg0, sg1, sg2, sw0, sw1, sw2):
    c = lax.axis_index("c")
    s = lax.axis_index("s")
    wid = s * 2 + c
    rbase = wid * RPW
    rj = [rj0, rj1, rj2]
    rw = [rw0, rw1, rw2]
    sg = [sg0, sg1, sg2]
    sw = [sw0, sw1, sw2]

    pltpu.sync_copy(vj_hbm.at[pl.ds(rbase, RPW)], idxj_v)
    pltpu.sync_copy(vw_hbm.at[pl.ds(rbase, RPW)], idxw_v)

    def fire_gather(b, ch):
        off = ch * CHUNK
        pltpu.async_copy(tabj_hbm.at[idxj_v.at[pl.ds(off, CHUNK)]], rj[b], sg[b])
        pltpu.async_copy(tabw_hbm.at[idxw_v.at[pl.ds(off, CHUNK)]], rw[b], sg[b])

    def wait_gather(b):
        pltpu.make_async_copy(tabj_hbm.at[idxj_v.at[pl.ds(0, CHUNK)]], rj[b], sg[b]).wait()
        pltpu.make_async_copy(tabw_hbm.at[idxw_v.at[pl.ds(0, CHUNK)]], rw[b], sg[b]).wait()

    def fire_wb(b, ch):
        gb = rbase + ch * CHUNK
        pltpu.async_copy(rj[b], outj_hbm.at[pl.ds(gb, CHUNK)], sw[b])
        pltpu.async_copy(rw[b], outw_hbm.at[pl.ds(gb, CHUNK)], sw[b])

    def wait_wb(b):
        pltpu.make_async_copy(rj[b], outj_hbm.at[pl.ds(0, CHUNK)], sw[b]).wait()
        pltpu.make_async_copy(rw[b], outw_hbm.at[pl.ds(0, CHUNK)], sw[b]).wait()

    for b in range(NBUF):
        fire_gather(b, b)

    def group(g, carry):
        ch0 = g * NBUF
        for b in range(NBUF):
            wait_gather(b)
            fire_wb(b, ch0 + b)
        for b in range(NBUF):
            @pl.when(g < GROUPS - 1)
            def _(b=b):
                wait_wb(b)
                fire_gather(b, ch0 + NBUF + b)
        return carry

    lax.fori_loop(0, GROUPS, group, 0)
    for b in range(NBUF):
        wait_wb(b)

    toff = NCH * CHUNK
    tj = rj[0].at[pl.ds(0, TAIL)]
    tw = rw[0].at[pl.ds(0, TAIL)]
    pltpu.async_copy(tabj_hbm.at[idxj_v.at[pl.ds(toff, TAIL)]], tj, sg0)
    pltpu.async_copy(tabw_hbm.at[idxw_v.at[pl.ds(toff, TAIL)]], tw, sg0)
    pltpu.make_async_copy(tabj_hbm.at[idxj_v.at[pl.ds(toff, TAIL)]], tj, sg0).wait()
    pltpu.make_async_copy(tabw_hbm.at[idxw_v.at[pl.ds(toff, TAIL)]], tw, sg0).wait()
    pltpu.sync_copy(tj, outj_hbm.at[pl.ds(rbase + toff, TAIL)])
    pltpu.sync_copy(tw, outw_hbm.at[pl.ds(rbase + toff, TAIL)])


def _sc_gather(tabj, tabw, vj, vw):
    return pl.kernel(
        _sc_gather_body,
        mesh=plsc.VectorSubcoreMesh(core_axis_name="c", subcore_axis_name="s"),
        compiler_params=pltpu.CompilerParams(use_tc_tiling_on_sc=False),
        out_type=(
            jax.ShapeDtypeStruct((NKALL, F), jnp.float32),
            jax.ShapeDtypeStruct((NKALL, DW), jnp.float32),
        ),
        scratch_types=[
            pltpu.VMEM((RPW,), jnp.int32),
            pltpu.VMEM((RPW,), jnp.int32),
            pltpu.VMEM((CHUNK, F), jnp.float32),
            pltpu.VMEM((CHUNK, F), jnp.float32),
            pltpu.VMEM((CHUNK, F), jnp.float32),
            pltpu.VMEM((CHUNK, DW), jnp.float32),
            pltpu.VMEM((CHUNK, DW), jnp.float32),
            pltpu.VMEM((CHUNK, DW), jnp.float32),
            pltpu.SemaphoreType.DMA,
            pltpu.SemaphoreType.DMA,
            pltpu.SemaphoreType.DMA,
            pltpu.SemaphoreType.DMA,
            pltpu.SemaphoreType.DMA,
            pltpu.SemaphoreType.DMA,
        ],
    )(tabj, tabw, vj, vw)


_B1 = 400


def _atten1_tc_body(ev_ref, ejn_ref, ewn_ref, w1e_ref, w1w_ref, w2_ref,
                    b_ref, v_ref, out_ref):
    bf = jnp.bfloat16
    f32 = jnp.float32
    ev = ev_ref[...]
    ejn = ejn_ref[...]
    ewn = ewn_ref[...]
    h = (jnp.dot(ev.astype(bf), w1e_ref[...].astype(bf),
                 preferred_element_type=f32) + b_ref[...])
    hj = jnp.dot(ejn.astype(bf), w2_ref[...].astype(bf),
                 preferred_element_type=f32)
    hw = jnp.dot(ewn.astype(bf), w1w_ref[...].astype(bf),
                 preferred_element_type=f32)
    av = (hj + hw).reshape(_B1, K, A) + h[:, None, :]
    x = jnp.sum(jnp.maximum(av, 0.0) * v_ref[...].reshape(1, 1, A), axis=-1)
    m = jnp.max(x, axis=1, keepdims=True)
    e = jnp.exp(x - m)
    a = e / jnp.sum(e, axis=1, keepdims=True)
    out_ref[...] = jnp.sum(a[:, :, None] * ejn.reshape(_B1, K, F), axis=1)


def _atten1_tc(call_idx, ev, ejn_all, ewn_all, w1e, w1w, w2, b, v):
    grid = (N // _B1,)
    boff = call_idx * (NK // (_B1 * K))
    return pl.pallas_call(
        _atten1_tc_body,
        grid=grid,
        in_specs=[
            pl.BlockSpec((_B1, F), lambda i: (i, 0)),
            pl.BlockSpec((_B1 * K, F), lambda i, o=boff: (o + i, 0)),
            pl.BlockSpec((_B1 * K, DW), lambda i, o=boff: (o + i, 0)),
            pl.BlockSpec((F, A), lambda i: (0, 0)),
            pl.BlockSpec((DW, A), lambda i: (0, 0)),
            pl.BlockSpec((F, A), lambda i: (0, 0)),
            pl.BlockSpec((1, A), lambda i: (0, 0)),
            pl.BlockSpec((1, A), lambda i: (0, 0)),
        ],
        out_specs=pl.BlockSpec((_B1, F), lambda i: (i, 0)),
        out_shape=jax.ShapeDtypeStruct((N, F), jnp.float32),
    )(ev, ejn_all, ewn_all, w1e, w1w, w2, b, v)


def _atten2_tc_body(u_ref, i_ref, t_ref, U_ref, q_ref, p_ref, out_ref):
    u = u_ref[...]
    i = i_ref[...]
    t = t_ref[...]
    Um = U_ref[...]
    q = q_ref[...]
    p = p_ref[...]
    xu = jnp.dot(u, Um, precision=_HI) + q
    xi = jnp.dot(i, Um, precision=_HI) + q
    xt = jnp.dot(t, Um, precision=_HI) + q
    su = jnp.sum(jnp.maximum(xu, 0.0) * p, axis=-1, keepdims=True)
    si = jnp.sum(jnp.maximum(xi, 0.0) * p, axis=-1, keepdims=True)
    st = jnp.sum(jnp.maximum(xt, 0.0) * p, axis=-1, keepdims=True)
    x = jnp.concatenate([su, si, st], axis=1)
    m = jnp.max(x, axis=1, keepdims=True)
    e = jnp.exp(x - m)
    a = e / jnp.sum(e, axis=1, keepdims=True)
    out_ref[...] = (a[:, 0:1] * u + a[:, 1:2] * i + a[:, 2:3] * t)


def _atten2_tc(u, i, t, U, q, p):
    grid = (N // _B1,)
    blk = pl.BlockSpec((_B1, F), lambda g: (g, 0))
    return pl.pallas_call(
        _atten2_tc_body,
        grid=grid,
        in_specs=[blk, blk, blk,
                  pl.BlockSpec((F, A), lambda g: (0, 0)),
                  pl.BlockSpec((1, A), lambda g: (0, 0)),
                  pl.BlockSpec((1, A), lambda g: (0, 0))],
        out_specs=blk,
        out_shape=jax.ShapeDtypeStruct((N, F), jnp.float32),
    )(u, i, t, U, q, p)


def kernel(eu, ei, et, ew, u_iw_j, u_iw_w, u_tw_j, u_tw_w, i_uw_j, i_uw_w,
           i_tw_j, i_tw_w, t_uw_j, t_uw_w, t_iw_j, t_iw_w, W1_user, W2_user,
           b_user, v_user, W1_item, W2_item, b_item, v_item, W1_tag, W2_tag,
           b_tag, v_tag, U, q, p):
    zrow = jnp.zeros((1, F), jnp.float32)
    tabj = jnp.concatenate([zrow, eu, zrow, ei, zrow, et], axis=0)
    tabw = jnp.concatenate([jnp.zeros((1, DW), jnp.float32), ew], axis=0)

    tmap = (1, 2, 0, 2, 0, 1)
    vjs = (u_iw_j, u_tw_j, i_uw_j, i_tw_j, t_uw_j, t_iw_j)
    vws = (u_iw_w, u_tw_w, i_uw_w, i_tw_w, t_uw_w, t_iw_w)
    vj_all = jnp.concatenate(
        [v.reshape(-1) + jnp.int32(tm * (N + 1)) for v, tm in zip(vjs, tmap)])
    vw_all = jnp.concatenate([v.reshape(-1) for v in vws])

    ejn_all, ewn_all = _sc_gather(tabj, tabw, vj_all, vw_all)

    def atten1(c, ev, W1, W2, b, v):
        return _atten1_tc(c, ev, ejn_all, ewn_all, W1[:F], W1[F:], W2, b, v)

    eu_iN = atten1(0, eu, W1_item, W2_item, b_item, v_item)
    eu_tN = atten1(1, eu, W1_tag, W2_tag, b_tag, v_tag)
    ei_uN = atten1(2, ei, W1_user, W2_user, b_user, v_user)
    ei_tN = atten1(3, ei, W1_tag, W2_tag, b_tag, v_tag)
    et_uN = atten1(4, et, W1_user, W2_user, b_user, v_user)
    et_iN = atten1(5, et, W1_item, W2_item, b_item, v_item)

    euN = _atten2_tc(eu, eu_iN, eu_tN, U, q, p)
    eiN = _atten2_tc(ei_uN, ei, ei_tN, U, q, p)
    etN = _atten2_tc(et_uN, et_iN, et, U, q, p)
    return (euN, eiN, etN)


# speedup vs baseline: 6.0223x; 1.0826x over previous
import jax
import jax.numpy as jnp
from jax import lax
from jax.experimental import pallas as pl
from jax.experimental.pallas import tpu as pltpu
from jax.experimental.pallas import tpu_sc as plsc

N = 10000
K = 16
F = 128
A = 128
DW = 16
NK = N * K
NCALLS = 6
NKALL = NCALLS * NK
NWORK = 32
CHUNK = 128
NBUF = 3

_HI = jax.lax.Precision.HIGHEST


def _make_sc_gather(nrows, width, untiled):
    rpw = nrows // NWORK
    nch = rpw // CHUNK
    tail = rpw - nch * CHUNK
    groups = nch // NBUF
    assert rpw * NWORK == nrows and groups * NBUF == nch and tail % 8 == 0

    def body(tab_hbm, idx_hbm, out_hbm, idx_v, r0, r1, r2, s0, s1, s2):
        c = lax.axis_index("c")
        s = lax.axis_index("s")
        wid = s * 2 + c
        rbase = wid * rpw
        rb = [r0, r1, r2]
        sm = [s0, s1, s2]

        pltpu.sync_copy(idx_hbm.at[pl.ds(rbase, rpw)], idx_v)

        def fire_gather(b, ch):
            pltpu.async_copy(
                tab_hbm.at[idx_v.at[pl.ds(ch * CHUNK, CHUNK)]], rb[b], sm[b])

        def wait_gather(b):
            pltpu.make_async_copy(
                tab_hbm.at[idx_v.at[pl.ds(0, CHUNK)]], rb[b], sm[b]).wait()

        def fire_wb(b, ch):
            pltpu.async_copy(
                rb[b], out_hbm.at[pl.ds(rbase + ch * CHUNK, CHUNK)], sm[b])

        def wait_wb(b):
            pltpu.make_async_copy(
                rb[b], out_hbm.at[pl.ds(0, CHUNK)], sm[b]).wait()

        for b in range(NBUF):
            fire_gather(b, b)

        def group(g, carry):
            ch0 = g * NBUF
            for b in range(NBUF):
                wait_gather(b)
                fire_wb(b, ch0 + b)
            for b in range(NBUF):
                @pl.when(g < groups - 1)
                def _(b=b):
                    wait_wb(b)
                    fire_gather(b, ch0 + NBUF + b)
            return carry

        lax.fori_loop(0, groups, group, 0)
        for b in range(NBUF):
            wait_wb(b)

        if tail:
            toff = nch * CHUNK
            t0 = r0.at[pl.ds(0, tail)]
            src = tab_hbm.at[idx_v.at[pl.ds(toff, tail)]]
            pltpu.async_copy(src, t0, s0)
            pltpu.make_async_copy(src, t0, s0).wait()
            pltpu.sync_copy(t0, out_hbm.at[pl.ds(rbase + toff, tail)])

    params = pltpu.CompilerParams(use_tc_tiling_on_sc=False) if untiled else None

    def run(tab, idx):
        return pl.kernel(
            body,
            mesh=plsc.VectorSubcoreMesh(core_axis_name="c", subcore_axis_name="s"),
            compiler_params=params,
            out_type=jax.ShapeDtypeStruct((nrows, width), jnp.float32),
            scratch_types=[
                pltpu.VMEM((rpw,), jnp.int32),
                pltpu.VMEM((CHUNK, width), jnp.float32),
                pltpu.VMEM((CHUNK, width), jnp.float32),
                pltpu.VMEM((CHUNK, width), jnp.float32),
                pltpu.SemaphoreType.DMA,
                pltpu.SemaphoreType.DMA,
                pltpu.SemaphoreType.DMA,
            ],
        )(tab, idx)

    return run


_gather_ej_half = _make_sc_gather(NKALL // 2, F, untiled=False)
_gather_ew_all = _make_sc_gather(NKALL, DW, untiled=True)


_B1 = 400


def _atten1_tc_body(ev_ref, ejn_ref, ewn_ref, w1e_ref, w1w_ref, w2_ref,
                    b_ref, v_ref, out_ref):
    bf = jnp.bfloat16
    f32 = jnp.float32
    ev = ev_ref[...]
    ejn = ejn_ref[...]
    ewn = ewn_ref[...]
    h = (jnp.dot(ev.astype(bf), w1e_ref[...].astype(bf),
                 preferred_element_type=f32) + b_ref[...])
    hj = jnp.dot(ejn.astype(bf), w2_ref[...].astype(bf),
                 preferred_element_type=f32)
    hw = jnp.dot(ewn.astype(bf), w1w_ref[...].astype(bf),
                 preferred_element_type=f32)
    av = (hj + hw).reshape(_B1, K, A) + h[:, None, :]
    x = jnp.sum(jnp.maximum(av, 0.0) * v_ref[...].reshape(1, 1, A), axis=-1)
    m = jnp.max(x, axis=1, keepdims=True)
    e = jnp.exp(x - m)
    a = e / jnp.sum(e, axis=1, keepdims=True)
    out_ref[...] = jnp.sum(a[:, :, None] * ejn.reshape(_B1, K, F), axis=1)


def _atten1_tc(call_idx, ev, ejn_half, ewn_all, w1e, w1w, w2, b, v):
    grid = (N // _B1,)
    nblk = NK // (_B1 * K)
    joff = (call_idx % 3) * nblk
    woff = call_idx * nblk
    return pl.pallas_call(
        _atten1_tc_body,
        grid=grid,
        in_specs=[
            pl.BlockSpec((_B1, F), lambda i: (i, 0)),
            pl.BlockSpec((_B1 * K, F), lambda i, o=joff: (o + i, 0)),
            pl.BlockSpec((_B1 * K, DW), lambda i, o=woff: (o + i, 0)),
            pl.BlockSpec((F, A), lambda i: (0, 0)),
            pl.BlockSpec((DW, A), lambda i: (0, 0)),
            pl.BlockSpec((F, A), lambda i: (0, 0)),
            pl.BlockSpec((1, A), lambda i: (0, 0)),
            pl.BlockSpec((1, A), lambda i: (0, 0)),
        ],
        out_specs=pl.BlockSpec((_B1, F), lambda i: (i, 0)),
        out_shape=jax.ShapeDtypeStruct((N, F), jnp.float32),
    )(ev, ejn_half, ewn_all, w1e, w1w, w2, b, v)


def _atten2_tc_body(u_ref, i_ref, t_ref, U_ref, q_ref, p_ref, out_ref):
    u = u_ref[...]
    i = i_ref[...]
    t = t_ref[...]
    Um = U_ref[...]
    q = q_ref[...]
    p = p_ref[...]
    xu = jnp.dot(u, Um, precision=_HI) + q
    xi = jnp.dot(i, Um, precision=_HI) + q
    xt = jnp.dot(t, Um, precision=_HI) + q
    su = jnp.sum(jnp.maximum(xu, 0.0) * p, axis=-1, keepdims=True)
    si = jnp.sum(jnp.maximum(xi, 0.0) * p, axis=-1, keepdims=True)
    st = jnp.sum(jnp.maximum(xt, 0.0) * p, axis=-1, keepdims=True)
    x = jnp.concatenate([su, si, st], axis=1)
    m = jnp.max(x, axis=1, keepdims=True)
    e = jnp.exp(x - m)
    a = e / jnp.sum(e, axis=1, keepdims=True)
    out_ref[...] = (a[:, 0:1] * u + a[:, 1:2] * i + a[:, 2:3] * t)


def _atten2_tc(u, i, t, U, q, p):
    grid = (N // _B1,)
    blk = pl.BlockSpec((_B1, F), lambda g: (g, 0))
    return pl.pallas_call(
        _atten2_tc_body,
        grid=grid,
        in_specs=[blk, blk, blk,
                  pl.BlockSpec((F, A), lambda g: (0, 0)),
                  pl.BlockSpec((1, A), lambda g: (0, 0)),
                  pl.BlockSpec((1, A), lambda g: (0, 0))],
        out_specs=blk,
        out_shape=jax.ShapeDtypeStruct((N, F), jnp.float32),
    )(u, i, t, U, q, p)


def kernel(eu, ei, et, ew, u_iw_j, u_iw_w, u_tw_j, u_tw_w, i_uw_j, i_uw_w,
           i_tw_j, i_tw_w, t_uw_j, t_uw_w, t_iw_j, t_iw_w, W1_user, W2_user,
           b_user, v_user, W1_item, W2_item, b_item, v_item, W1_tag, W2_tag,
           b_tag, v_tag, U, q, p):
    zrow = jnp.zeros((1, F), jnp.float32)
    tabj = jnp.concatenate([zrow, eu, zrow, ei, zrow, et], axis=0)
    tabw = jnp.concatenate([jnp.zeros((1, DW), jnp.float32), ew], axis=0)

    tmap = (1, 2, 0, 2, 0, 1)
    vjs = (u_iw_j, u_tw_j, i_uw_j, i_tw_j, t_uw_j, t_iw_j)
    vws = (u_iw_w, u_tw_w, i_uw_w, i_tw_w, t_uw_w, t_iw_w)
    vj_off = [v.reshape(-1) + jnp.int32(tm * (N + 1)) for v, tm in zip(vjs, tmap)]
    vw_all = jnp.concatenate([v.reshape(-1) for v in vws])

    ewn_all = _gather_ew_all(tabw, vw_all)
    ejn_h0 = _gather_ej_half(tabj, jnp.concatenate(vj_off[:3]))
    ejn_h1 = _gather_ej_half(tabj, jnp.concatenate(vj_off[3:]))
    ejn_halves = (ejn_h0, ejn_h1)

    def atten1(c, ev, W1, W2, b, v):
        return _atten1_tc(c, ev, ejn_halves[c // 3], ewn_all,
                          W1[:F], W1[F:], W2, b, v)

    eu_iN = atten1(0, eu, W1_item, W2_item, b_item, v_item)
    eu_tN = atten1(1, eu, W1_tag, W2_tag, b_tag, v_tag)
    ei_uN = atten1(2, ei, W1_user, W2_user, b_user, v_user)
    ei_tN = atten1(3, ei, W1_tag, W2_tag, b_tag, v_tag)
    et_uN = atten1(4, et, W1_user, W2_user, b_user, v_user)
    et_iN = atten1(5, et, W1_item, W2_item, b_item, v_item)

    euN = _atten2_tc(eu, eu_iN, eu_tN, U, q, p)
    eiN = _atten2_tc(ei_uN, ei, ei_tN, U, q, p)
    etN = _atten2_tc(et_uN, et_iN, et, U, q, p)
    return (euN, eiN, etN)
